# baseline (speedup 1.0000x reference)
import functools

import numpy as np

import jax
import jax.numpy as jnp
from jax import lax
from jax.experimental import pallas as pl
from jax.experimental.pallas import tpu as pltpu
from jax.experimental.pallas import tpu_sc as plsc

_N, _F, _D, _C, _E = 10000, 128, 32, 16, 320000
_NC, _NS = 2, 16
_NW = _NC * _NS
_EPW = _E // _NW
_CE = 1250
_NCH = _EPW // _CE
_NP = 10240
_NPT = _NP // _NS
_RC = 128
_ZCH = _NPT // _RC
_BN_S = float(1.0 / np.sqrt(1.0 + 1e-5))
_PREC = lax.Precision.DEFAULT


def _sc_agg_body(table, srcr, dstr, out, src_v, dst_v, rows0, rows1, zbuf,
                 sem0, sem1, acc):
    c = lax.axis_index("c")
    s = lax.axis_index("s")
    wid = s * _NC + c
    idx_cp0 = pltpu.make_async_copy(srcr.at[wid], src_v, sem0)
    idx_cp1 = pltpu.make_async_copy(dstr.at[wid], dst_v, sem1)
    idx_cp0.start()
    idx_cp1.start()

    def zstore(i, _):
        zbuf[i, pl.ds(0, 16)] = jnp.zeros((16,), jnp.float32)
        zbuf[i, pl.ds(16, 16)] = jnp.zeros((16,), jnp.float32)
        return 0

    lax.fori_loop(0, _RC, zstore, 0)
    for z in range(_ZCH):
        pltpu.sync_copy(zbuf, acc.at[pl.ds(s * _NPT + z * _RC, _RC)])
    idx_cp0.wait()
    idx_cp1.wait()
    plsc.subcore_barrier()

    def g_start(j, buf, sem):
        pltpu.make_async_copy(table.at[src_v.at[j]], buf, sem).start()

    def g_wait(j, buf, sem):
        pltpu.make_async_copy(table.at[src_v.at[j]], buf, sem).wait()

    def scat(j, buf):
        pltpu.sync_copy(buf, acc.at[dst_v.at[j]], add=True)

    g_start(0, rows0, sem0)
    g_start(1, rows1, sem1)

    def body(t, _):
        j0 = 2 * t
        j1 = 2 * t + 1
        g_wait(j0, rows0, sem0)
        scat(j0, rows0)
        g_start(j0 + 2, rows0, sem0)
        g_wait(j1, rows1, sem1)
        scat(j1, rows1)
        g_start(j1 + 2, rows1, sem1)
        return 0

    lax.fori_loop(0, _NCH // 2 - 1, body, 0)
    g_wait(_NCH - 2, rows0, sem0)
    scat(_NCH - 2, rows0)
    g_wait(_NCH - 1, rows1, sem1)
    scat(_NCH - 1, rows1)

    plsc.subcore_barrier()
    pltpu.sync_copy(acc.at[pl.ds(s * _NPT, _NPT)],
                    out.at[c, pl.ds(s * _NPT, _NPT)])


@functools.lru_cache(maxsize=None)
def _make_sc_agg():
    mesh = plsc.VectorSubcoreMesh(core_axis_name="c", subcore_axis_name="s")
    return functools.partial(
        pl.kernel,
        mesh=mesh,
        compiler_params=pltpu.CompilerParams(use_tc_tiling_on_sc=False),
        out_type=jax.ShapeDtypeStruct((_NC, _NP, _D), jnp.float32),
        scratch_types=[
            pltpu.VMEM((_NCH, _CE), jnp.int32),
            pltpu.VMEM((_NCH, _CE), jnp.int32),
            pltpu.VMEM((_CE, _D), jnp.float32),
            pltpu.VMEM((_CE, _D), jnp.float32),
            pltpu.VMEM((_RC, _D), jnp.float32),
            pltpu.SemaphoreType.DMA,
            pltpu.SemaphoreType.DMA,
            pltpu.VMEM_SHARED((_NP, _D), jnp.float32),
        ],
    )(_sc_agg_body)


_NP4 = _NP // 4
_N4 = _N // 4


def _mm1_body(x_ref, w_ref, o_ref):
    w = w_ref[...]
    for k in range(3):
        o_ref[:, pl.ds(k * _D, _D)] = jnp.dot(
            x_ref[pl.ds(k * _NP4, _NP4), :], w,
            preferred_element_type=jnp.float32, precision=_PREC)
    nlast = _N - 3 * _NP4
    o_ref[pl.ds(0, nlast), pl.ds(3 * _D, _D)] = jnp.dot(
        x_ref[pl.ds(3 * _NP4, nlast), :], w,
        preferred_element_type=jnp.float32, precision=_PREC)


def _mid_body(p_ref, agg_ref, b1a_ref, w1b_ref, b1b_ref, g1_ref, be1_ref,
              w2a_ref, q_ref):
    aggs = agg_ref[...]
    h = p_ref[...] + aggs[0] + aggs[1] + b1a_ref[...]
    h = jnp.maximum(h, 0.0)
    h = jnp.dot(h, w1b_ref[...], preferred_element_type=jnp.float32,
                precision=_PREC) + b1b_ref[...]
    h = jnp.maximum(h, 0.0)
    t = h * (_BN_S * g1_ref[...]) + be1_ref[...]
    q_ref[...] = jnp.dot(t, w2a_ref[...], preferred_element_type=jnp.float32,
                         precision=_PREC)


def _fin_body(q_ref, agg_ref, b2a_ref, w2b_ref, b2b_ref, g2_ref, be2_ref,
              wf1_ref, bf1_ref, wf2_ref, bf2_ref, o_ref):
    aggs = agg_ref[...]
    h = q_ref[...] + aggs[0] + aggs[1] + b2a_ref[...]
    h = jnp.maximum(h, 0.0)
    h = jnp.maximum(
        jnp.dot(h, w2b_ref[...], preferred_element_type=jnp.float32,
                precision=_PREC) + b2b_ref[...], 0.0)
    u = h * (_BN_S * g2_ref[...]) + be2_ref[...]
    h = jnp.maximum(
        jnp.dot(u, wf1_ref[...], preferred_element_type=jnp.float32,
                precision=_PREC) + bf1_ref[...], 0.0)
    z4 = jnp.dot(h, wf2_ref[...], preferred_element_type=jnp.float32,
                 precision=_PREC) + bf2_ref[...]
    nlast = _N - 3 * _NP4
    for k in range(4):
        zb = z4[:, k * _C:(k + 1) * _C]
        m = jnp.max(zb, axis=-1, keepdims=True)
        e = zb - m
        lsb = e - jnp.log(jnp.sum(jnp.exp(e), axis=-1, keepdims=True))
        if k < 3:
            o_ref[pl.ds(k * _NP4, _NP4), :] = lsb
        else:
            o_ref[pl.ds(3 * _NP4, nlast), :] = lsb[:nlast]


def _bd4(w):
    return jnp.kron(jnp.eye(4, dtype=w.dtype), w)


def kernel(x, edge_index, W1a, b1a, W1b, b1b, g1, be1, W2a, b2a, W2b, b2b,
           g2, be2, Wf1, bf1, Wf2, bf2):
    perm = (edge_index % _NP4) * 4 + edge_index // _NP4
    srcr = perm[0].reshape(_NW, _NCH, _CE)
    dstr = perm[1].reshape(_NW, _NCH, _CE)
    sc_agg = _make_sc_agg()

    p4 = pl.pallas_call(
        _mm1_body,
        out_shape=jax.ShapeDtypeStruct((_NP4, 4 * _D), jnp.float32))(x, W1a)
    agg1 = sc_agg(p4.reshape(_NP, _D), srcr, dstr)
    q4 = pl.pallas_call(
        _mid_body,
        out_shape=jax.ShapeDtypeStruct((_NP4, 4 * _D), jnp.float32))(
            p4, agg1.reshape(_NC, _NP4, 4 * _D),
            jnp.tile(b1a, 4).reshape(1, 4 * _D), _bd4(W1b),
            jnp.tile(b1b, 4).reshape(1, 4 * _D),
            jnp.tile(g1, 4).reshape(1, 4 * _D),
            jnp.tile(be1, 4).reshape(1, 4 * _D), _bd4(W2a))
    agg2 = sc_agg(q4.reshape(_NP, _D), srcr, dstr)
    out = pl.pallas_call(
        _fin_body,
        out_shape=jax.ShapeDtypeStruct((_N, _C), jnp.float32))(
            q4, agg2.reshape(_NC, _NP4, 4 * _D),
            jnp.tile(b2a, 4).reshape(1, 4 * _D), _bd4(W2b),
            jnp.tile(b2b, 4).reshape(1, 4 * _D),
            jnp.tile(g2, 4).reshape(1, 4 * _D),
            jnp.tile(be2, 4).reshape(1, 4 * _D), _bd4(Wf1),
            jnp.tile(bf1, 4).reshape(1, 4 * _D), _bd4(Wf2),
            jnp.tile(bf2, 4).reshape(1, 4 * _C))
    return out


# speedup vs baseline: 1.0645x; 1.0645x over previous
import functools

import numpy as np

import jax
import jax.numpy as jnp
from jax import lax
from jax.experimental import pallas as pl
from jax.experimental.pallas import tpu as pltpu
from jax.experimental.pallas import tpu_sc as plsc

_N, _F, _D, _C, _E = 10000, 128, 32, 16, 320000
_NC, _NS = 2, 16
_NW = _NC * _NS
_EPW = _E // _NW
_CE = 1000
_NCH = _EPW // _CE
_NP = 10240
_NPT = _NP // _NS
_RC = 128
_ZCH = _NPT // _RC
_BN_S = float(1.0 / np.sqrt(1.0 + 1e-5))
_PREC = lax.Precision.DEFAULT


def _sc_agg_body(table, srcr, dstr, out, src_v, dst_v, rows0, rows1, zbuf,
                 sem0, sem1, acc):
    c = lax.axis_index("c")
    s = lax.axis_index("s")
    wid = s * _NC + c
    idx_cp0 = pltpu.make_async_copy(srcr.at[wid], src_v, sem0)
    idx_cp1 = pltpu.make_async_copy(dstr.at[wid], dst_v, sem1)
    idx_cp0.start()
    idx_cp1.start()

    def zstore(i, _):
        zbuf[i, pl.ds(0, 16)] = jnp.zeros((16,), jnp.float32)
        zbuf[i, pl.ds(16, 16)] = jnp.zeros((16,), jnp.float32)
        return 0

    lax.fori_loop(0, _RC, zstore, 0)
    for z in range(_ZCH):
        pltpu.sync_copy(zbuf, acc.at[pl.ds(s * _NPT + z * _RC, _RC)])
    idx_cp0.wait()
    idx_cp1.wait()
    plsc.subcore_barrier()

    def g_start(j, buf, sem):
        pltpu.make_async_copy(table.at[src_v.at[j]], buf, sem).start()

    def g_wait(j, buf, sem):
        pltpu.make_async_copy(table.at[src_v.at[j]], buf, sem).wait()

    def scat(j, buf):
        pltpu.sync_copy(buf, acc.at[dst_v.at[j]], add=True)

    g_start(0, rows0, sem0)
    g_start(1, rows1, sem1)

    def body(t, _):
        j0 = 2 * t
        j1 = 2 * t + 1
        g_wait(j0, rows0, sem0)
        scat(j0, rows0)
        g_start(j0 + 2, rows0, sem0)
        g_wait(j1, rows1, sem1)
        scat(j1, rows1)
        g_start(j1 + 2, rows1, sem1)
        return 0

    lax.fori_loop(0, _NCH // 2 - 1, body, 0)
    g_wait(_NCH - 2, rows0, sem0)
    scat(_NCH - 2, rows0)
    g_wait(_NCH - 1, rows1, sem1)
    scat(_NCH - 1, rows1)

    plsc.subcore_barrier()
    pltpu.sync_copy(acc.at[pl.ds(s * _NPT, _NPT)],
                    out.at[c, pl.ds(s * _NPT, _NPT)])


@functools.lru_cache(maxsize=None)
def _make_sc_agg():
    mesh = plsc.VectorSubcoreMesh(core_axis_name="c", subcore_axis_name="s")
    return functools.partial(
        pl.kernel,
        mesh=mesh,
        compiler_params=pltpu.CompilerParams(use_tc_tiling_on_sc=False),
        out_type=jax.ShapeDtypeStruct((_NC, _NP, _D), jnp.float32),
        scratch_types=[
            pltpu.VMEM((_NCH, _CE), jnp.int32),
            pltpu.VMEM((_NCH, _CE), jnp.int32),
            pltpu.VMEM((_CE, _D), jnp.float32),
            pltpu.VMEM((_CE, _D), jnp.float32),
            pltpu.VMEM((_RC, _D), jnp.float32),
            pltpu.SemaphoreType.DMA,
            pltpu.SemaphoreType.DMA,
            pltpu.VMEM_SHARED((_NP, _D), jnp.float32),
        ],
    )(_sc_agg_body)


_NP4 = _NP // 4
_N4 = _N // 4


def _mm1_body(x_ref, w_ref, o_ref):
    w = w_ref[...]
    for k in range(3):
        o_ref[:, pl.ds(k * _D, _D)] = jnp.dot(
            x_ref[pl.ds(k * _NP4, _NP4), :], w,
            preferred_element_type=jnp.float32, precision=_PREC)
    nlast = _N - 3 * _NP4
    o_ref[pl.ds(0, nlast), pl.ds(3 * _D, _D)] = jnp.dot(
        x_ref[pl.ds(3 * _NP4, nlast), :], w,
        preferred_element_type=jnp.float32, precision=_PREC)


def _mid_body(p_ref, agg_ref, b1a_ref, w1b_ref, b1b_ref, g1_ref, be1_ref,
              w2a_ref, q_ref):
    aggs = agg_ref[...]
    h = p_ref[...] + aggs[0] + aggs[1] + b1a_ref[...]
    h = jnp.maximum(h, 0.0)
    h = jnp.dot(h, w1b_ref[...], preferred_element_type=jnp.float32,
                precision=_PREC) + b1b_ref[...]
    h = jnp.maximum(h, 0.0)
    t = h * (_BN_S * g1_ref[...]) + be1_ref[...]
    q_ref[...] = jnp.dot(t, w2a_ref[...], preferred_element_type=jnp.float32,
                         precision=_PREC)


def _fin_body(q_ref, agg_ref, b2a_ref, w2b_ref, b2b_ref, g2_ref, be2_ref,
              wf1_ref, bf1_ref, wf2_ref, bf2_ref, o_ref):
    aggs = agg_ref[...]
    h = q_ref[...] + aggs[0] + aggs[1] + b2a_ref[...]
    h = jnp.maximum(h, 0.0)
    h = jnp.maximum(
        jnp.dot(h, w2b_ref[...], preferred_element_type=jnp.float32,
                precision=_PREC) + b2b_ref[...], 0.0)
    u = h * (_BN_S * g2_ref[...]) + be2_ref[...]
    h = jnp.maximum(
        jnp.dot(u, wf1_ref[...], preferred_element_type=jnp.float32,
                precision=_PREC) + bf1_ref[...], 0.0)
    z4 = jnp.dot(h, wf2_ref[...], preferred_element_type=jnp.float32,
                 precision=_PREC) + bf2_ref[...]
    nlast = _N - 3 * _NP4
    for k in range(4):
        zb = z4[:, k * _C:(k + 1) * _C]
        m = jnp.max(zb, axis=-1, keepdims=True)
        e = zb - m
        lsb = e - jnp.log(jnp.sum(jnp.exp(e), axis=-1, keepdims=True))
        if k < 3:
            o_ref[pl.ds(k * _NP4, _NP4), :] = lsb
        else:
            o_ref[pl.ds(3 * _NP4, nlast), :] = lsb[:nlast]


def _bd4(w):
    return jnp.kron(jnp.eye(4, dtype=w.dtype), w)


def kernel(x, edge_index, W1a, b1a, W1b, b1b, g1, be1, W2a, b2a, W2b, b2b,
           g2, be2, Wf1, bf1, Wf2, bf2):
    perm = (edge_index % _NP4) * 4 + edge_index // _NP4
    srcr = perm[0].reshape(_NW, _NCH, _CE)
    dstr = perm[1].reshape(_NW, _NCH, _CE)
    sc_agg = _make_sc_agg()

    p4 = pl.pallas_call(
        _mm1_body,
        out_shape=jax.ShapeDtypeStruct((_NP4, 4 * _D), jnp.float32))(x, W1a)
    agg1 = sc_agg(p4.reshape(_NP, _D), srcr, dstr)
    q4 = pl.pallas_call(
        _mid_body,
        out_shape=jax.ShapeDtypeStruct((_NP4, 4 * _D), jnp.float32))(
            p4, agg1.reshape(_NC, _NP4, 4 * _D),
            jnp.tile(b1a, 4).reshape(1, 4 * _D), _bd4(W1b),
            jnp.tile(b1b, 4).reshape(1, 4 * _D),
            jnp.tile(g1, 4).reshape(1, 4 * _D),
            jnp.tile(be1, 4).reshape(1, 4 * _D), _bd4(W2a))
    agg2 = sc_agg(q4.reshape(_NP, _D), srcr, dstr)
    out = pl.pallas_call(
        _fin_body,
        out_shape=jax.ShapeDtypeStruct((_N, _C), jnp.float32))(
            q4, agg2.reshape(_NC, _NP4, 4 * _D),
            jnp.tile(b2a, 4).reshape(1, 4 * _D), _bd4(W2b),
            jnp.tile(b2b, 4).reshape(1, 4 * _D),
            jnp.tile(g2, 4).reshape(1, 4 * _D),
            jnp.tile(be2, 4).reshape(1, 4 * _D), _bd4(Wf1),
            jnp.tile(bf1, 4).reshape(1, 4 * _D), _bd4(Wf2),
            jnp.tile(bf2, 4).reshape(1, 4 * _C))
    return out
